# Initial kernel scaffold; baseline (speedup 1.0000x reference)
import jax
import jax.numpy as jnp
from jax.experimental import pallas as pl


def kernel(x, edge_index, W1_l, W1_r, b1, W2_l, W2_r, b2):
    raise NotImplementedError("write your pallas kernel here")


# speedup vs baseline: 6.7661x; 6.7661x over previous
import functools

import jax
import jax.numpy as jnp
from jax import lax
from jax.experimental import pallas as pl
from jax.experimental.pallas import tpu as pltpu
from jax.experimental.pallas import tpu_sc as plsc

N_NODES = 10000
N_EDGES = 320000
D = 128

NC = 2
NS = 16
NW = NC * NS
EPW = N_EDGES // NW
CH = 128
NFULL = EPW // CH
TAIL = EPW - NFULL * CH
ROWS_PT = 624
ROWS_LAST = N_NODES - (NS - 1) * ROWS_PT
CNT_PT = 624


@functools.cache
def _make_seg_sum(with_cnt: bool):
  mesh = plsc.VectorSubcoreMesh(
      core_axis_name="c", subcore_axis_name="s", num_cores=NC,
      num_subcores=NS)

  out_type = [jax.ShapeDtypeStruct((NC, N_NODES, D), jnp.float32)]
  if with_cnt:
    out_type.append(jax.ShapeDtypeStruct((NC * N_NODES,), jnp.float32))

  scratch = dict(
      src_v=pltpu.VMEM((CH,), jnp.int32),
      dst_v=pltpu.VMEM((CH,), jnp.int32),
      rows_v=pltpu.VMEM((CH, D), jnp.float32),
      src_t=pltpu.VMEM((TAIL,), jnp.int32),
      dst_t=pltpu.VMEM((TAIL,), jnp.int32),
      rows_t=pltpu.VMEM((TAIL, D), jnp.float32),
      ones_v=pltpu.VMEM((CH,), jnp.float32),
      cbuf=pltpu.VMEM((ROWS_LAST,), jnp.float32),
      acc_sp=pltpu.VMEM_SHARED((N_NODES, D), jnp.float32),
      cnt_sp=pltpu.VMEM_SHARED((N_NODES,), jnp.float32),
      sem=pltpu.SemaphoreType.DMA,
  )

  def body(src_hbm, dst_hbm, feat_hbm, z2d_hbm, *outs, src_v, dst_v,
           rows_v, src_t, dst_t, rows_t, ones_v, cbuf, acc_sp, cnt_sp, sem):
    if with_cnt:
      acc_out, cnt_out = outs
    else:
      (acc_out,) = outs

    cid = lax.axis_index("c")
    sid = lax.axis_index("s")
    wid = cid * NS + sid

    @pl.when(sid < NS - 1)
    def _():
      pltpu.sync_copy(z2d_hbm.at[pl.ds(sid * ROWS_PT, ROWS_PT)],
                      acc_sp.at[pl.ds(sid * ROWS_PT, ROWS_PT)])

    @pl.when(sid == NS - 1)
    def _():
      pltpu.sync_copy(z2d_hbm.at[pl.ds((NS - 1) * ROWS_PT, ROWS_LAST)],
                      acc_sp.at[pl.ds((NS - 1) * ROWS_PT, ROWS_LAST)])

    if with_cnt:
      for j in range(CH // 16):
        ones_v[pl.ds(j * 16, 16)] = jnp.ones((16,), jnp.float32)
      for j in range(ROWS_LAST // 16):
        cbuf[pl.ds(j * 16, 16)] = jnp.zeros((16,), jnp.float32)

      @pl.when(sid < NS - 1)
      def _():
        pltpu.sync_copy(cbuf.at[pl.ds(0, CNT_PT)],
                        cnt_sp.at[pl.ds(sid * CNT_PT, CNT_PT)])

      @pl.when(sid == NS - 1)
      def _():
        nlast = N_NODES - (NS - 1) * CNT_PT
        pltpu.sync_copy(cbuf.at[pl.ds(0, nlast)],
                        cnt_sp.at[pl.ds((NS - 1) * CNT_PT, nlast)])

    plsc.subcore_barrier()

    ebase = wid * EPW

    def chunk(i, _):
      base = ebase + i * CH
      pltpu.sync_copy(src_hbm.at[pl.ds(base, CH)], src_v)
      pltpu.sync_copy(dst_hbm.at[pl.ds(base, CH)], dst_v)
      pltpu.async_copy(feat_hbm.at[src_v], rows_v, sem).wait()
      pltpu.sync_copy(rows_v, acc_sp.at[dst_v], add=True)
      if with_cnt:
        pltpu.sync_copy(ones_v, cnt_sp.at[dst_v], add=True)
      return 0

    lax.fori_loop(0, NFULL, chunk, 0)

    tbase = ebase + NFULL * CH
    pltpu.sync_copy(src_hbm.at[pl.ds(tbase, TAIL)], src_t)
    pltpu.sync_copy(dst_hbm.at[pl.ds(tbase, TAIL)], dst_t)
    pltpu.async_copy(feat_hbm.at[src_t], rows_t, sem).wait()
    pltpu.sync_copy(rows_t, acc_sp.at[dst_t], add=True)
    if with_cnt:
      pltpu.sync_copy(ones_v.at[pl.ds(0, TAIL)], cnt_sp.at[dst_t], add=True)

    plsc.subcore_barrier()

    @pl.when(sid < NS - 1)
    def _():
      pltpu.sync_copy(acc_sp.at[pl.ds(sid * ROWS_PT, ROWS_PT)],
                      acc_out.at[cid, pl.ds(sid * ROWS_PT, ROWS_PT)])

    @pl.when(sid == NS - 1)
    def _():
      pltpu.sync_copy(acc_sp.at[pl.ds((NS - 1) * ROWS_PT, ROWS_LAST)],
                      acc_out.at[cid, pl.ds((NS - 1) * ROWS_PT, ROWS_LAST)])

    if with_cnt:
      @pl.when(sid < NS - 1)
      def _():
        pltpu.sync_copy(cnt_sp.at[pl.ds(sid * CNT_PT, CNT_PT)],
                        cbuf.at[pl.ds(0, CNT_PT)])
        pltpu.sync_copy(
            cbuf.at[pl.ds(0, CNT_PT)],
            cnt_out.at[pl.ds(cid * N_NODES + sid * CNT_PT, CNT_PT)])

      @pl.when(sid == NS - 1)
      def _():
        nlast = N_NODES - (NS - 1) * CNT_PT
        pltpu.sync_copy(cnt_sp.at[pl.ds((NS - 1) * CNT_PT, nlast)],
                        cbuf.at[pl.ds(0, nlast)])
        pltpu.sync_copy(
            cbuf.at[pl.ds(0, nlast)],
            cnt_out.at[pl.ds(cid * N_NODES + (NS - 1) * CNT_PT, nlast)])

  return pl.kernel(body, out_type=out_type, mesh=mesh,
                   scratch_types=scratch,
                   name="seg_sum_cnt" if with_cnt else "seg_sum")


RB = 1000
GRID = N_NODES // RB


def _dense_body(acc_ref, cnt_ref, x_ref, wl_ref, wr_ref, b_ref, o_ref, *,
                final: bool):
  s = acc_ref[0] + acc_ref[1]
  c = cnt_ref[0] + cnt_ref[1]
  mean = s / jnp.maximum(c, 1.0)
  z = (jnp.dot(mean, wl_ref[...], preferred_element_type=jnp.float32)
       + jnp.dot(x_ref[...], wr_ref[...], preferred_element_type=jnp.float32)
       + b_ref[...])
  if final:
    m = jnp.max(z, axis=1, keepdims=True)
    e = jnp.exp(z - m)
    lse = jnp.log(jnp.sum(e, axis=1, keepdims=True)) + m
    o_ref[...] = z - lse
  else:
    o_ref[...] = jnp.maximum(z, 0.0)


def _dense(acc, cnt, x, w_l, w_r, b, final: bool):
  return pl.pallas_call(
      functools.partial(_dense_body, final=final),
      grid=(GRID,),
      in_specs=[
          pl.BlockSpec((NC, RB, D), lambda i: (0, i, 0)),
          pl.BlockSpec((NC, RB, 1), lambda i: (0, i, 0)),
          pl.BlockSpec((RB, D), lambda i: (i, 0)),
          pl.BlockSpec((D, D), lambda i: (0, 0)),
          pl.BlockSpec((D, D), lambda i: (0, 0)),
          pl.BlockSpec((D,), lambda i: (0,)),
      ],
      out_specs=pl.BlockSpec((RB, D), lambda i: (i, 0)),
      out_shape=jax.ShapeDtypeStruct((N_NODES, D), jnp.float32),
  )(acc, cnt, x, w_l, w_r, b)


def kernel(x, edge_index, W1_l, W1_r, b1, W2_l, W2_r, b2):
  ei = edge_index.astype(jnp.int32)
  src = ei[0]
  dst = ei[1]
  z2d = jnp.zeros((N_NODES, D), jnp.float32)

  acc1, cnt = _make_seg_sum(True)(src, dst, x, z2d)
  cnt3 = cnt.reshape(NC, N_NODES, 1)
  h = _dense(acc1, cnt3, x, W1_l, W1_r, b1, final=False)
  (acc2,) = _make_seg_sum(False)(src, dst, h, z2d)
  out = _dense(acc2, cnt3, h, W2_l, W2_r, b2, final=True)
  return out


# speedup vs baseline: 11.8945x; 1.7580x over previous
import functools

import jax
import jax.numpy as jnp
from jax import lax
from jax.experimental import pallas as pl
from jax.experimental.pallas import tpu as pltpu
from jax.experimental.pallas import tpu_sc as plsc

N_NODES = 10000
N_EDGES = 320000
D = 128

NC = 2
NS = 16
NW = NC * NS
EPW = N_EDGES // NW
CH = 128
NFULL = EPW // CH
TAIL = EPW - NFULL * CH
ROWS_PT = 624
ROWS_LAST = N_NODES - (NS - 1) * ROWS_PT
CNT_PT = 624


@functools.cache
def _make_seg_sum(with_cnt: bool):
  mesh = plsc.VectorSubcoreMesh(
      core_axis_name="c", subcore_axis_name="s", num_cores=NC,
      num_subcores=NS)

  out_type = [jax.ShapeDtypeStruct((NC, N_NODES, D), jnp.float32)]
  if with_cnt:
    out_type.append(jax.ShapeDtypeStruct((NC * N_NODES,), jnp.float32))

  scratch = dict(
      src0=pltpu.VMEM((CH,), jnp.int32),
      dst0=pltpu.VMEM((CH,), jnp.int32),
      src1=pltpu.VMEM((CH,), jnp.int32),
      dst1=pltpu.VMEM((CH,), jnp.int32),
      rows0=pltpu.VMEM((CH, D), jnp.float32),
      rows1=pltpu.VMEM((CH, D), jnp.float32),
      src_t=pltpu.VMEM((TAIL,), jnp.int32),
      dst_t=pltpu.VMEM((TAIL,), jnp.int32),
      rows_t=pltpu.VMEM((TAIL, D), jnp.float32),
      ones_v=pltpu.VMEM((CH,), jnp.float32),
      cbuf=pltpu.VMEM((ROWS_LAST,), jnp.float32),
      acc_sp=pltpu.VMEM_SHARED((N_NODES, D), jnp.float32),
      cnt_sp=pltpu.VMEM_SHARED((N_NODES,), jnp.float32),
      sem_i0=pltpu.SemaphoreType.DMA,
      sem_i1=pltpu.SemaphoreType.DMA,
      sem_g0=pltpu.SemaphoreType.DMA,
      sem_g1=pltpu.SemaphoreType.DMA,
      sem=pltpu.SemaphoreType.DMA,
  )

  def body(src_hbm, dst_hbm, feat_hbm, z2d_hbm, *outs, src0, dst0, src1,
           dst1, rows0, rows1, src_t, dst_t, rows_t, ones_v, cbuf, acc_sp,
           cnt_sp, sem_i0, sem_i1, sem_g0, sem_g1, sem):
    if with_cnt:
      acc_out, cnt_out = outs
    else:
      (acc_out,) = outs

    cid = lax.axis_index("c")
    sid = lax.axis_index("s")
    wid = cid * NS + sid

    @pl.when(sid < NS - 1)
    def _():
      pltpu.sync_copy(z2d_hbm.at[pl.ds(sid * ROWS_PT, ROWS_PT)],
                      acc_sp.at[pl.ds(sid * ROWS_PT, ROWS_PT)])

    @pl.when(sid == NS - 1)
    def _():
      pltpu.sync_copy(z2d_hbm.at[pl.ds((NS - 1) * ROWS_PT, ROWS_LAST)],
                      acc_sp.at[pl.ds((NS - 1) * ROWS_PT, ROWS_LAST)])

    if with_cnt:
      for j in range(CH // 16):
        ones_v[pl.ds(j * 16, 16)] = jnp.ones((16,), jnp.float32)
      for j in range(ROWS_LAST // 16):
        cbuf[pl.ds(j * 16, 16)] = jnp.zeros((16,), jnp.float32)

      @pl.when(sid < NS - 1)
      def _():
        pltpu.sync_copy(cbuf.at[pl.ds(0, CNT_PT)],
                        cnt_sp.at[pl.ds(sid * CNT_PT, CNT_PT)])

      @pl.when(sid == NS - 1)
      def _():
        nlast = N_NODES - (NS - 1) * CNT_PT
        pltpu.sync_copy(cbuf.at[pl.ds(0, nlast)],
                        cnt_sp.at[pl.ds((NS - 1) * CNT_PT, nlast)])

    plsc.subcore_barrier()

    ebase = wid * EPW
    bufs = ((src0, dst0, rows0, sem_i0, sem_g0),
            (src1, dst1, rows1, sem_i1, sem_g1))

    def cbase(i):
      return jnp.minimum(ebase + i * CH, N_EDGES - CH)

    def idx_start(i, b):
      sv, dv, _, si, _ = bufs[b]
      base = cbase(i)
      pltpu.async_copy(src_hbm.at[pl.ds(base, CH)], sv, si)
      pltpu.async_copy(dst_hbm.at[pl.ds(base, CH)], dv, si)

    def idx_wait(b):
      sv, dv, _, si, _ = bufs[b]
      pltpu.make_async_copy(src_hbm.at[pl.ds(0, CH)], sv, si).wait()
      pltpu.make_async_copy(dst_hbm.at[pl.ds(0, CH)], dv, si).wait()

    def gather_start(b):
      sv, _, rv, _, sg = bufs[b]
      pltpu.async_copy(feat_hbm.at[sv], rv, sg)

    def gather_wait(b):
      sv, _, rv, _, sg = bufs[b]
      pltpu.make_async_copy(feat_hbm.at[sv], rv, sg).wait()

    def step(i, b):
      nb = 1 - b
      idx_wait(nb)
      gather_start(nb)
      gather_wait(b)
      _, dv, rv, _, _ = bufs[b]
      pltpu.sync_copy(rv, acc_sp.at[dv], add=True)
      if with_cnt:
        pltpu.sync_copy(ones_v, cnt_sp.at[dv], add=True)
      idx_start(i + 2, b)

    idx_start(0, 0)
    idx_wait(0)
    gather_start(0)
    idx_start(1, 1)

    def pair(j, _):
      step(2 * j, 0)
      step(2 * j + 1, 1)
      return 0

    lax.fori_loop(0, NFULL // 2, pair, 0)

    gather_wait(0)
    idx_wait(1)

    tbase = ebase + NFULL * CH
    pltpu.sync_copy(src_hbm.at[pl.ds(tbase, TAIL)], src_t)
    pltpu.sync_copy(dst_hbm.at[pl.ds(tbase, TAIL)], dst_t)
    pltpu.async_copy(feat_hbm.at[src_t], rows_t, sem).wait()
    pltpu.sync_copy(rows_t, acc_sp.at[dst_t], add=True)
    if with_cnt:
      pltpu.sync_copy(ones_v.at[pl.ds(0, TAIL)], cnt_sp.at[dst_t], add=True)

    plsc.subcore_barrier()

    @pl.when(sid < NS - 1)
    def _():
      pltpu.sync_copy(acc_sp.at[pl.ds(sid * ROWS_PT, ROWS_PT)],
                      acc_out.at[cid, pl.ds(sid * ROWS_PT, ROWS_PT)])

    @pl.when(sid == NS - 1)
    def _():
      pltpu.sync_copy(acc_sp.at[pl.ds((NS - 1) * ROWS_PT, ROWS_LAST)],
                      acc_out.at[cid, pl.ds((NS - 1) * ROWS_PT, ROWS_LAST)])

    if with_cnt:
      @pl.when(sid < NS - 1)
      def _():
        pltpu.sync_copy(cnt_sp.at[pl.ds(sid * CNT_PT, CNT_PT)],
                        cbuf.at[pl.ds(0, CNT_PT)])
        pltpu.sync_copy(
            cbuf.at[pl.ds(0, CNT_PT)],
            cnt_out.at[pl.ds(cid * N_NODES + sid * CNT_PT, CNT_PT)])

      @pl.when(sid == NS - 1)
      def _():
        nlast = N_NODES - (NS - 1) * CNT_PT
        pltpu.sync_copy(cnt_sp.at[pl.ds((NS - 1) * CNT_PT, nlast)],
                        cbuf.at[pl.ds(0, nlast)])
        pltpu.sync_copy(
            cbuf.at[pl.ds(0, nlast)],
            cnt_out.at[pl.ds(cid * N_NODES + (NS - 1) * CNT_PT, nlast)])

  return pl.kernel(body, out_type=out_type, mesh=mesh,
                   scratch_types=scratch,
                   name="seg_sum_cnt" if with_cnt else "seg_sum")


RB = 1000
GRID = N_NODES // RB


def _dense_body(acc_ref, cnt_ref, x_ref, wl_ref, wr_ref, b_ref, o_ref, *,
                final: bool):
  s = acc_ref[0] + acc_ref[1]
  c = cnt_ref[0] + cnt_ref[1]
  mean = s / jnp.maximum(c, 1.0)
  z = (jnp.dot(mean, wl_ref[...], preferred_element_type=jnp.float32)
       + jnp.dot(x_ref[...], wr_ref[...], preferred_element_type=jnp.float32)
       + b_ref[...])
  if final:
    m = jnp.max(z, axis=1, keepdims=True)
    e = jnp.exp(z - m)
    lse = jnp.log(jnp.sum(e, axis=1, keepdims=True)) + m
    o_ref[...] = z - lse
  else:
    o_ref[...] = jnp.maximum(z, 0.0)


def _dense(acc, cnt, x, w_l, w_r, b, final: bool):
  return pl.pallas_call(
      functools.partial(_dense_body, final=final),
      grid=(GRID,),
      in_specs=[
          pl.BlockSpec((NC, RB, D), lambda i: (0, i, 0)),
          pl.BlockSpec((NC, RB, 1), lambda i: (0, i, 0)),
          pl.BlockSpec((RB, D), lambda i: (i, 0)),
          pl.BlockSpec((D, D), lambda i: (0, 0)),
          pl.BlockSpec((D, D), lambda i: (0, 0)),
          pl.BlockSpec((D,), lambda i: (0,)),
      ],
      out_specs=pl.BlockSpec((RB, D), lambda i: (i, 0)),
      out_shape=jax.ShapeDtypeStruct((N_NODES, D), jnp.float32),
  )(acc, cnt, x, w_l, w_r, b)


def kernel(x, edge_index, W1_l, W1_r, b1, W2_l, W2_r, b2):
  ei = edge_index.astype(jnp.int32)
  src = ei[0]
  dst = ei[1]
  z2d = jnp.zeros((N_NODES, D), jnp.float32)

  acc1, cnt = _make_seg_sum(True)(src, dst, x, z2d)
  cnt3 = cnt.reshape(NC, N_NODES, 1)
  h = _dense(acc1, cnt3, x, W1_l, W1_r, b1, final=False)
  (acc2,) = _make_seg_sum(False)(src, dst, h, z2d)
  out = _dense(acc2, cnt3, h, W2_l, W2_r, b2, final=True)
  return out


# speedup vs baseline: 12.0301x; 1.0114x over previous
import functools

import jax
import jax.numpy as jnp
from jax import lax
from jax.experimental import pallas as pl
from jax.experimental.pallas import tpu as pltpu
from jax.experimental.pallas import tpu_sc as plsc

N_NODES = 10000
N_EDGES = 320000
D = 128

NC = 2
NS = 16
NW = NC * NS
EPW = N_EDGES // NW
CH = 80
NFULL = EPW // CH
ROWS_PT = 624
ROWS_LAST = N_NODES - (NS - 1) * ROWS_PT
CNT_PT = 624


@functools.cache
def _make_seg_sum(with_cnt: bool):
  mesh = plsc.VectorSubcoreMesh(
      core_axis_name="c", subcore_axis_name="s", num_cores=NC,
      num_subcores=NS)

  out_type = [jax.ShapeDtypeStruct((NC, N_NODES, D), jnp.float32)]
  if with_cnt:
    out_type.append(jax.ShapeDtypeStruct((NC * N_NODES,), jnp.float32))

  NB = 4
  scratch = dict(
      srcs=[pltpu.VMEM((CH,), jnp.int32) for _ in range(NB)],
      dsts=[pltpu.VMEM((CH,), jnp.int32) for _ in range(NB)],
      rows=[pltpu.VMEM((CH, D), jnp.float32) for _ in range(NB)],
      ones_v=pltpu.VMEM((CH,), jnp.float32),
      cbuf=pltpu.VMEM((ROWS_LAST,), jnp.float32),
      acc_sp=pltpu.VMEM_SHARED((N_NODES, D), jnp.float32),
      cnt_sp=pltpu.VMEM_SHARED((N_NODES,), jnp.float32),
      sem_i=[pltpu.SemaphoreType.DMA for _ in range(NB)],
      sem_g=[pltpu.SemaphoreType.DMA for _ in range(NB)],
      sem_s=[pltpu.SemaphoreType.DMA for _ in range(NB)],
      sem_c=[pltpu.SemaphoreType.DMA for _ in range(NB)],
  )

  def body(src_hbm, dst_hbm, feat_hbm, z2d_hbm, *outs, srcs, dsts, rows,
           ones_v, cbuf, acc_sp, cnt_sp, sem_i, sem_g, sem_s, sem_c):
    if with_cnt:
      acc_out, cnt_out = outs
    else:
      (acc_out,) = outs

    cid = lax.axis_index("c")
    sid = lax.axis_index("s")
    wid = cid * NS + sid

    @pl.when(sid < NS - 1)
    def _():
      pltpu.sync_copy(z2d_hbm.at[pl.ds(sid * ROWS_PT, ROWS_PT)],
                      acc_sp.at[pl.ds(sid * ROWS_PT, ROWS_PT)])

    @pl.when(sid == NS - 1)
    def _():
      pltpu.sync_copy(z2d_hbm.at[pl.ds((NS - 1) * ROWS_PT, ROWS_LAST)],
                      acc_sp.at[pl.ds((NS - 1) * ROWS_PT, ROWS_LAST)])

    if with_cnt:
      for j in range(CH // 16):
        ones_v[pl.ds(j * 16, 16)] = jnp.ones((16,), jnp.float32)
      for j in range(ROWS_LAST // 16):
        cbuf[pl.ds(j * 16, 16)] = jnp.zeros((16,), jnp.float32)

      @pl.when(sid < NS - 1)
      def _():
        pltpu.sync_copy(cbuf.at[pl.ds(0, CNT_PT)],
                        cnt_sp.at[pl.ds(sid * CNT_PT, CNT_PT)])

      @pl.when(sid == NS - 1)
      def _():
        nlast = N_NODES - (NS - 1) * CNT_PT
        pltpu.sync_copy(cbuf.at[pl.ds(0, nlast)],
                        cnt_sp.at[pl.ds((NS - 1) * CNT_PT, nlast)])

    plsc.subcore_barrier()

    ebase = wid * EPW
    NB = len(rows)

    def cbase(i):
      return jnp.minimum(ebase + i * CH, N_EDGES - CH)

    def idx_start(i, b):
      base = cbase(i)
      pltpu.async_copy(src_hbm.at[pl.ds(base, CH)], srcs[b], sem_i[b])
      pltpu.async_copy(dst_hbm.at[pl.ds(base, CH)], dsts[b], sem_i[b])

    def idx_wait(b):
      pltpu.make_async_copy(src_hbm.at[pl.ds(0, CH)], srcs[b],
                            sem_i[b]).wait()
      pltpu.make_async_copy(dst_hbm.at[pl.ds(0, CH)], dsts[b],
                            sem_i[b]).wait()

    def gather_start(b):
      pltpu.async_copy(feat_hbm.at[srcs[b]], rows[b], sem_g[b])

    def gather_wait(b):
      pltpu.make_async_copy(feat_hbm.at[srcs[b]], rows[b], sem_g[b]).wait()

    def scatter_start(b):
      pltpu.async_copy(rows[b], acc_sp.at[dsts[b]], sem_s[b], add=True)
      if with_cnt:
        pltpu.async_copy(ones_v, cnt_sp.at[dsts[b]], sem_c[b], add=True)

    def scatter_wait(b):
      pltpu.make_async_copy(rows[b], acc_sp.at[dsts[b]], sem_s[b]).wait()
      if with_cnt:
        pltpu.make_async_copy(ones_v, cnt_sp.at[dsts[b]], sem_c[b]).wait()

    def step(i, b, first=False):
      nb = (b + 1) % NB
      fb = (b + 2) % NB
      if not first:
        scatter_wait(fb)
      idx_wait(nb)
      gather_start(nb)
      gather_wait(b)
      scatter_start(b)
      idx_start(i + 2, fb)

    idx_start(0, 0)
    idx_wait(0)
    gather_start(0)
    idx_start(1, 1)

    step(0, 0, first=True)
    step(1, 1, first=True)

    def quad(j, _):
      i0 = 4 * j + 2
      for k in range(4):
        step(i0 + k, (2 + k) % NB)
      return 0

    n_quads = (NFULL - 6) // 4
    lax.fori_loop(0, n_quads, quad, 0)

    for i in range(2 + 4 * n_quads, NFULL):
      step(i, i % NB)

    scatter_wait((NFULL - 2) % NB)
    scatter_wait((NFULL - 1) % NB)
    gather_wait(NFULL % NB)
    idx_wait((NFULL + 1) % NB)

    plsc.subcore_barrier()

    @pl.when(sid < NS - 1)
    def _():
      pltpu.sync_copy(acc_sp.at[pl.ds(sid * ROWS_PT, ROWS_PT)],
                      acc_out.at[cid, pl.ds(sid * ROWS_PT, ROWS_PT)])

    @pl.when(sid == NS - 1)
    def _():
      pltpu.sync_copy(acc_sp.at[pl.ds((NS - 1) * ROWS_PT, ROWS_LAST)],
                      acc_out.at[cid, pl.ds((NS - 1) * ROWS_PT, ROWS_LAST)])

    if with_cnt:
      @pl.when(sid < NS - 1)
      def _():
        pltpu.sync_copy(cnt_sp.at[pl.ds(sid * CNT_PT, CNT_PT)],
                        cbuf.at[pl.ds(0, CNT_PT)])
        pltpu.sync_copy(
            cbuf.at[pl.ds(0, CNT_PT)],
            cnt_out.at[pl.ds(cid * N_NODES + sid * CNT_PT, CNT_PT)])

      @pl.when(sid == NS - 1)
      def _():
        nlast = N_NODES - (NS - 1) * CNT_PT
        pltpu.sync_copy(cnt_sp.at[pl.ds((NS - 1) * CNT_PT, nlast)],
                        cbuf.at[pl.ds(0, nlast)])
        pltpu.sync_copy(
            cbuf.at[pl.ds(0, nlast)],
            cnt_out.at[pl.ds(cid * N_NODES + (NS - 1) * CNT_PT, nlast)])

  return pl.kernel(body, out_type=out_type, mesh=mesh,
                   scratch_types=scratch,
                   name="seg_sum_cnt" if with_cnt else "seg_sum")


RB = 1000
GRID = N_NODES // RB


def _dense_body(acc_ref, cnt_ref, x_ref, wl_ref, wr_ref, b_ref, o_ref, *,
                final: bool):
  s = acc_ref[0] + acc_ref[1]
  c = cnt_ref[0] + cnt_ref[1]
  mean = s / jnp.maximum(c, 1.0)
  z = (jnp.dot(mean, wl_ref[...], preferred_element_type=jnp.float32)
       + jnp.dot(x_ref[...], wr_ref[...], preferred_element_type=jnp.float32)
       + b_ref[...])
  if final:
    m = jnp.max(z, axis=1, keepdims=True)
    e = jnp.exp(z - m)
    lse = jnp.log(jnp.sum(e, axis=1, keepdims=True)) + m
    o_ref[...] = z - lse
  else:
    o_ref[...] = jnp.maximum(z, 0.0)


def _dense(acc, cnt, x, w_l, w_r, b, final: bool):
  return pl.pallas_call(
      functools.partial(_dense_body, final=final),
      grid=(GRID,),
      in_specs=[
          pl.BlockSpec((NC, RB, D), lambda i: (0, i, 0)),
          pl.BlockSpec((NC, RB, 1), lambda i: (0, i, 0)),
          pl.BlockSpec((RB, D), lambda i: (i, 0)),
          pl.BlockSpec((D, D), lambda i: (0, 0)),
          pl.BlockSpec((D, D), lambda i: (0, 0)),
          pl.BlockSpec((D,), lambda i: (0,)),
      ],
      out_specs=pl.BlockSpec((RB, D), lambda i: (i, 0)),
      out_shape=jax.ShapeDtypeStruct((N_NODES, D), jnp.float32),
  )(acc, cnt, x, w_l, w_r, b)


def kernel(x, edge_index, W1_l, W1_r, b1, W2_l, W2_r, b2):
  ei = edge_index.astype(jnp.int32)
  src = ei[0]
  dst = ei[1]
  z2d = jnp.zeros((N_NODES, D), jnp.float32)

  acc1, cnt = _make_seg_sum(True)(src, dst, x, z2d)
  cnt3 = cnt.reshape(NC, N_NODES, 1)
  h = _dense(acc1, cnt3, x, W1_l, W1_r, b1, final=False)
  (acc2,) = _make_seg_sum(False)(src, dst, h, z2d)
  out = _dense(acc2, cnt3, h, W2_l, W2_r, b2, final=True)
  return out
